# baseline (speedup 1.0000x reference)
import functools

import jax
import jax.numpy as jnp
from jax import lax
from jax.experimental import pallas as pl
from jax.experimental.pallas import tpu as pltpu
from jax.experimental.pallas import tpu_sc as plsc

NW = 32
NC = 2


def _elu(x):
    return jnp.where(x > 0, x, jnp.exp(x) - 1.0)


def _sc_gather(ent, nbt, num, idx_act, idx_sml):
    NE, SD = ent.shape
    NACT = idx_act.shape[0]
    NSML = idx_sml.shape[0]
    per_a = NACT // NW
    per_s = NSML // NW
    CA = 320
    CS = 64

    mesh = plsc.VectorSubcoreMesh(core_axis_name="c", subcore_axis_name="s")

    @functools.partial(
        pl.kernel,
        out_type=[
            jax.ShapeDtypeStruct((NACT, SD), jnp.float32),
            jax.ShapeDtypeStruct((NSML, SD), jnp.float32),
            jax.ShapeDtypeStruct((NSML, SD), jnp.float32),
            jax.ShapeDtypeStruct((NSML * 8,), jnp.float32),
        ],
        mesh=mesh,
        compiler_params=pltpu.CompilerParams(needs_layout_passes=False),
        scratch_types=[
            pltpu.VMEM((2, CA, SD), jnp.float32),
            pltpu.VMEM((2, CS, SD), jnp.float32),
            pltpu.VMEM((2, CS * 8), jnp.float32),
            pltpu.VMEM((per_a,), jnp.int32),
            pltpu.VMEM((per_s,), jnp.int32),
            pltpu.SemaphoreType.DMA,
            pltpu.SemaphoreType.DMA,
            pltpu.SemaphoreType.DMA,
        ],
    )
    def k(ent_hbm, nbt_hbm, num_hbm, idxa_hbm, idxs_hbm,
          out_a, out_e, out_n, out_q,
          astage, sstage, nstage, aidx, sidx, semg0, semg1, sems):
        wid = lax.axis_index("s") * NC + lax.axis_index("c")
        abase = wid * per_a
        sbase = wid * per_s
        pltpu.sync_copy(idxa_hbm.at[pl.ds(abase, per_a)], aidx)
        pltpu.sync_copy(idxs_hbm.at[pl.ds(sbase, per_s)], sidx)
        iota16 = lax.iota(jnp.int32, 16)
        gsem = (semg0, semg1)

        def phase(src, idxref, outref, hbase, nch, cp, stg, seg):
            rw = 8 if seg else src.shape[1] if len(src.shape) > 1 else 1

            def fire(g, b):
                def fire1(i, carry):
                    j16 = pl.multiple_of(
                        jnp.bitwise_and(g * cp + i, -16), 16)
                    lane = jnp.bitwise_and(i, 15)
                    vec = idxref[pl.ds(j16, 16)]
                    r = jnp.sum(jnp.where(iota16 == lane, vec, 0))
                    if seg:
                        r8 = pl.multiple_of(jnp.bitwise_and(r, -8), 8)
                        pltpu.async_copy(
                            src.at[pl.ds(r8, 8)],
                            stg.at[b, pl.ds(i * 8, 8)], gsem[b])
                    else:
                        pltpu.async_copy(
                            src.at[pl.ds(r, 1)],
                            stg.at[b, pl.ds(i, 1)], gsem[b])
                    return carry

                lax.fori_loop(0, cp, fire1, 0)

            def drain(b):
                pltpu.make_async_copy(
                    src.at[pl.ds(0, cp)] if not seg
                    else src.at[pl.ds(0, cp * 8)], stg.at[b], gsem[b]).wait()

            def store(g, b):
                if seg:
                    dst = outref.at[pl.ds((hbase + g * cp) * 8, cp * 8)]
                else:
                    dst = outref.at[pl.ds(hbase + g * cp, cp)]
                pltpu.async_copy(stg.at[b], dst, sems)
                return cp * rw * 4

            def drain_store():
                pltpu.make_async_copy(
                    src.at[pl.ds(0, cp)] if not seg
                    else src.at[pl.ds(0, cp * 8)], stg.at[0], sems).wait()

            def pair(h, carry):
                g0 = h * 2
                fire(g0, 0)
                fire(g0 + 1, 1)
                drain(0)
                store(g0, 0)
                drain(1)
                store(g0 + 1, 1)
                drain_store()
                drain_store()
                return carry

            lax.fori_loop(0, nch // 2, pair, 0)

        phase(ent_hbm, aidx, out_a, abase, per_a // CA, CA, astage, 0)
        phase(ent_hbm, sidx, out_e, sbase, per_s // CS, CS, sstage, 0)
        phase(nbt_hbm, sidx, out_n, sbase, per_s // CS, CS, sstage, 0)
        phase(num_hbm, sidx, out_q, sbase, per_s // CS, CS, nstage, 1)

    return k(ent, nbt, num, idx_act, idx_sml)


def _prep_body(title_ref, e13_ref, nb2_ref, nq_ref, col_ref, W1_ref, b1_ref,
               W2_ref, b2_ref, wan_ref, wae_ref, ba1_ref, s_out, sim_out,
               *, Bp, K):
    title = title_ref[...]
    h = _elu(jnp.dot(title, W1_ref[...], preferred_element_type=jnp.float32)
             + b1_ref[...])
    news = jnp.tanh(jnp.dot(h, W2_ref[...], preferred_element_type=jnp.float32)
                    + b2_ref[...])
    me = jnp.mean(e13_ref[...], axis=1)
    s_out[...] = (jnp.dot(news, wan_ref[...], preferred_element_type=jnp.float32)
                  + jnp.dot(me, wae_ref[...], preferred_element_type=jnp.float32)
                  + ba1_ref[...])
    D = news.shape[1]
    news_exp = jnp.broadcast_to(news[:, None, :], (Bp, K, D)).reshape(Bp * K, D)
    nb = nb2_ref[...]
    diff = nb - news_exp
    dots = jnp.sum(diff * news_exp, axis=-1)
    v2 = jnp.sum(diff * diff, axis=-1)
    n2 = jnp.sum(news_exp * news_exp, axis=-1)
    cols = col_ref[...]
    onehot = (lax.broadcasted_iota(jnp.int32, (Bp * K, 8), 1)
              == cols[:, None]).astype(jnp.float32)
    nnum = jnp.sum(nq_ref[...] * onehot, axis=-1)
    na = jnp.sqrt(n2)
    nbn = jnp.sqrt(v2) / nnum
    sim_out[...] = (dots / nnum) / jnp.maximum(na * nbn, 1e-8)


def _big_body(act_ref, s_ref, wa1a_ref, w2c_ref, b2c_ref, w3c_ref, b3c_ref,
              pq_ref, *, Bb, KK, D):
    a = act_ref[...].reshape(Bb * KK, D)
    z = jnp.dot(a, wa1a_ref[...], preferred_element_type=jnp.float32)
    z = z.reshape(Bb, KK, D) + s_ref[...][:, None, :]
    ax = _elu(z).reshape(Bb * KK, D)
    uv = _elu(jnp.dot(ax, w2c_ref[...], preferred_element_type=jnp.float32)
              + b2c_ref[...])
    pq_ref[...] = jax.nn.sigmoid(
        jnp.dot(uv, w3c_ref[...], preferred_element_type=jnp.float32)
        + b3c_ref[...])


def kernel(title_emb, entity_ids, neighbor_ids, entity_table, neibor_table,
           neibor_num, W1, b1, W2, b2, Wa1, ba1, Wa2, ba2, Wa3, ba3, Wc2, bc2,
           Wc3, bc3):
    B, K = entity_ids.shape
    KK = K * K
    D = entity_table.shape[1]

    eflat = entity_ids.reshape(-1).astype(jnp.int32)
    nflat = neighbor_ids.reshape(-1).astype(jnp.int32)

    act_rows, e1_rows, nb_rows, nnum_g = _sc_gather(
        entity_table, neibor_table, neibor_num, nflat, eflat)

    wan, wae, wa1a = Wa1[:D], Wa1[D:2 * D], Wa1[2 * D:]

    Bp = 256
    T = title_emb.shape[1]
    e13 = e1_rows.reshape(B, K, D)
    s_state, sim_flat = pl.pallas_call(
        functools.partial(_prep_body, Bp=Bp, K=K),
        grid=(B // Bp,),
        in_specs=[
            pl.BlockSpec((Bp, T), lambda i: (i, 0)),
            pl.BlockSpec((Bp, K, D), lambda i: (i, 0, 0)),
            pl.BlockSpec((Bp * K, D), lambda i: (i, 0)),
            pl.BlockSpec((Bp * K, 8), lambda i: (i, 0)),
            pl.BlockSpec((Bp * K,), lambda i: (i,)),
            pl.BlockSpec((T, D), lambda i: (0, 0)),
            pl.BlockSpec((1, D), lambda i: (0, 0)),
            pl.BlockSpec((D, D), lambda i: (0, 0)),
            pl.BlockSpec((1, D), lambda i: (0, 0)),
            pl.BlockSpec((D, D), lambda i: (0, 0)),
            pl.BlockSpec((D, D), lambda i: (0, 0)),
            pl.BlockSpec((1, D), lambda i: (0, 0)),
        ],
        out_specs=[
            pl.BlockSpec((Bp, D), lambda i: (i, 0)),
            pl.BlockSpec((Bp * K,), lambda i: (i,)),
        ],
        out_shape=[jax.ShapeDtypeStruct((B, D), jnp.float32),
                   jax.ShapeDtypeStruct((B * K,), jnp.float32)],
    )(title_emb, e13, nb_rows, nnum_g.reshape(B * K, 8),
      jnp.bitwise_and(eflat, 7), W1, b1.reshape(1, D), W2,
      b2.reshape(1, D), wan, wae, ba1.reshape(1, D))

    w2c = jnp.concatenate([Wa2, Wc2], axis=1)
    b2c = jnp.concatenate([ba2, bc2]).reshape(1, 2 * D)
    zD1 = jnp.zeros((D, 1), jnp.float32)
    w3c = jnp.concatenate(
        [jnp.concatenate([Wa3, zD1], axis=1),
         jnp.concatenate([zD1, Wc3], axis=1)], axis=0)
    b3c = jnp.concatenate([ba3, bc3]).reshape(1, 2)

    Bb = 32
    act3 = act_rows.reshape(B, KK, D)
    pq = pl.pallas_call(
        functools.partial(_big_body, Bb=Bb, KK=KK, D=D),
        grid=(B // Bb,),
        in_specs=[
            pl.BlockSpec((Bb, KK, D), lambda i: (i, 0, 0)),
            pl.BlockSpec((Bb, D), lambda i: (i, 0)),
            pl.BlockSpec((D, D), lambda i: (0, 0)),
            pl.BlockSpec((D, 2 * D), lambda i: (0, 0)),
            pl.BlockSpec((1, 2 * D), lambda i: (0, 0)),
            pl.BlockSpec((2 * D, 2), lambda i: (0, 0)),
            pl.BlockSpec((1, 2), lambda i: (0, 0)),
        ],
        out_specs=pl.BlockSpec((Bb * KK, 2), lambda i: (i, 0)),
        out_shape=jax.ShapeDtypeStruct((B * KK, 2), jnp.float32),
    )(act3, s_state, wa1a, w2c, b2c, w3c, b3c)

    return (pq[:, 0:1].reshape(B, KK, 1), pq[:, 1:2].reshape(B, KK, 1),
            sim_flat.reshape(B, K))


# speedup vs baseline: 1.1721x; 1.1721x over previous
import functools

import jax
import jax.numpy as jnp
from jax import lax
from jax.experimental import pallas as pl
from jax.experimental.pallas import tpu as pltpu
from jax.experimental.pallas import tpu_sc as plsc

NW = 32
NC = 2


def _elu(x):
    return jnp.where(x > 0, x, jnp.exp(x) - 1.0)


def _sc_gather(ent, nbt, num, idx_act, idx_sml):
    NE, SD = ent.shape
    NACT = idx_act.shape[0]
    NSML = idx_sml.shape[0]
    per_a = NACT // NW
    per_s = NSML // NW
    CA = 320
    CS = 64

    mesh = plsc.VectorSubcoreMesh(core_axis_name="c", subcore_axis_name="s")

    @functools.partial(
        pl.kernel,
        out_type=[
            jax.ShapeDtypeStruct((NACT, SD), jnp.float32),
            jax.ShapeDtypeStruct((NSML, SD), jnp.float32),
            jax.ShapeDtypeStruct((NSML, SD), jnp.float32),
            jax.ShapeDtypeStruct((NSML * 8,), jnp.float32),
        ],
        mesh=mesh,
        compiler_params=pltpu.CompilerParams(needs_layout_passes=False),
        scratch_types=[
            pltpu.VMEM((2, CA, SD), jnp.float32),
            pltpu.VMEM((2, CS, SD), jnp.float32),
            pltpu.VMEM((2, CS * 8), jnp.float32),
            pltpu.VMEM((per_a,), jnp.int32),
            pltpu.VMEM((per_s,), jnp.int32),
            pltpu.SemaphoreType.DMA,
            pltpu.SemaphoreType.DMA,
            pltpu.SemaphoreType.DMA,
        ],
    )
    def k(ent_hbm, nbt_hbm, num_hbm, idxa_hbm, idxs_hbm,
          out_a, out_e, out_n, out_q,
          astage, sstage, nstage, aidx, sidx, semg0, semg1, sems):
        wid = lax.axis_index("s") * NC + lax.axis_index("c")
        abase = wid * per_a
        sbase = wid * per_s
        pltpu.sync_copy(idxa_hbm.at[pl.ds(abase, per_a)], aidx)
        pltpu.sync_copy(idxs_hbm.at[pl.ds(sbase, per_s)], sidx)
        iota16 = lax.iota(jnp.int32, 16)
        gsem = (semg0, semg1)

        def phase(src, idxref, outref, hbase, nch, cp, stg, seg):
            rw = 8 if seg else src.shape[1] if len(src.shape) > 1 else 1

            def fire(g, b):
                def fire16(j, carry):
                    j16 = pl.multiple_of(g * cp + j * 16, 16)
                    vec = idxref[pl.ds(j16, 16)]
                    for l in range(16):
                        r = jnp.sum(jnp.where(iota16 == l, vec, 0))
                        if seg:
                            r8 = pl.multiple_of(jnp.bitwise_and(r, -8), 8)
                            pltpu.async_copy(
                                src.at[pl.ds(r8, 8)],
                                stg.at[b, pl.ds((j * 16 + l) * 8, 8)],
                                gsem[b])
                        else:
                            pltpu.async_copy(
                                src.at[pl.ds(r, 1)],
                                stg.at[b, pl.ds(j * 16 + l, 1)], gsem[b])
                    return carry

                lax.fori_loop(0, cp // 16, fire16, 0)

            def drain(b):
                pltpu.make_async_copy(
                    src.at[pl.ds(0, cp)] if not seg
                    else src.at[pl.ds(0, cp * 8)], stg.at[b], gsem[b]).wait()

            def store(g, b):
                if seg:
                    dst = outref.at[pl.ds((hbase + g * cp) * 8, cp * 8)]
                else:
                    dst = outref.at[pl.ds(hbase + g * cp, cp)]
                pltpu.async_copy(stg.at[b], dst, sems)
                return cp * rw * 4

            def drain_store():
                pltpu.make_async_copy(
                    src.at[pl.ds(0, cp)] if not seg
                    else src.at[pl.ds(0, cp * 8)], stg.at[0], sems).wait()

            def pair(h, carry):
                g0 = h * 2
                fire(g0, 0)
                fire(g0 + 1, 1)
                drain(0)
                store(g0, 0)
                drain(1)
                store(g0 + 1, 1)
                drain_store()
                drain_store()
                return carry

            lax.fori_loop(0, nch // 2, pair, 0)

        phase(ent_hbm, aidx, out_a, abase, per_a // CA, CA, astage, 0)
        phase(ent_hbm, sidx, out_e, sbase, per_s // CS, CS, sstage, 0)
        phase(nbt_hbm, sidx, out_n, sbase, per_s // CS, CS, sstage, 0)
        phase(num_hbm, sidx, out_q, sbase, per_s // CS, CS, nstage, 1)

    return k(ent, nbt, num, idx_act, idx_sml)


def _prep_body(title_ref, e13_ref, nb2_ref, nq_ref, col_ref, W1_ref, b1_ref,
               W2_ref, b2_ref, wan_ref, wae_ref, ba1_ref, s_out, sim_out,
               *, Bp, K):
    title = title_ref[...]
    h = _elu(jnp.dot(title, W1_ref[...], preferred_element_type=jnp.float32)
             + b1_ref[...])
    news = jnp.tanh(jnp.dot(h, W2_ref[...], preferred_element_type=jnp.float32)
                    + b2_ref[...])
    me = jnp.mean(e13_ref[...], axis=1)
    s_out[...] = (jnp.dot(news, wan_ref[...], preferred_element_type=jnp.float32)
                  + jnp.dot(me, wae_ref[...], preferred_element_type=jnp.float32)
                  + ba1_ref[...])
    D = news.shape[1]
    news_exp = jnp.broadcast_to(news[:, None, :], (Bp, K, D)).reshape(Bp * K, D)
    nb = nb2_ref[...]
    diff = nb - news_exp
    dots = jnp.sum(diff * news_exp, axis=-1)
    v2 = jnp.sum(diff * diff, axis=-1)
    n2 = jnp.sum(news_exp * news_exp, axis=-1)
    cols = col_ref[...]
    onehot = (lax.broadcasted_iota(jnp.int32, (Bp * K, 8), 1)
              == cols[:, None]).astype(jnp.float32)
    nnum = jnp.sum(nq_ref[...] * onehot, axis=-1)
    na = jnp.sqrt(n2)
    nbn = jnp.sqrt(v2) / nnum
    sim_out[...] = (dots / nnum) / jnp.maximum(na * nbn, 1e-8)


def _big_body(act_ref, s_ref, wa1a_ref, w2c_ref, b2c_ref, w3c_ref, b3c_ref,
              pq_ref, *, Bb, KK, D):
    a = act_ref[...].reshape(Bb * KK, D)
    z = jnp.dot(a, wa1a_ref[...], preferred_element_type=jnp.float32)
    z = z.reshape(Bb, KK, D) + s_ref[...][:, None, :]
    ax = _elu(z).reshape(Bb * KK, D)
    uv = _elu(jnp.dot(ax, w2c_ref[...], preferred_element_type=jnp.float32)
              + b2c_ref[...])
    pq_ref[...] = jax.nn.sigmoid(
        jnp.dot(uv, w3c_ref[...], preferred_element_type=jnp.float32)
        + b3c_ref[...])


def kernel(title_emb, entity_ids, neighbor_ids, entity_table, neibor_table,
           neibor_num, W1, b1, W2, b2, Wa1, ba1, Wa2, ba2, Wa3, ba3, Wc2, bc2,
           Wc3, bc3):
    B, K = entity_ids.shape
    KK = K * K
    D = entity_table.shape[1]

    eflat = entity_ids.reshape(-1).astype(jnp.int32)
    nflat = neighbor_ids.reshape(-1).astype(jnp.int32)

    act_rows, e1_rows, nb_rows, nnum_g = _sc_gather(
        entity_table, neibor_table, neibor_num, nflat, eflat)

    wan, wae, wa1a = Wa1[:D], Wa1[D:2 * D], Wa1[2 * D:]

    Bp = 256
    T = title_emb.shape[1]
    e13 = e1_rows.reshape(B, K, D)
    s_state, sim_flat = pl.pallas_call(
        functools.partial(_prep_body, Bp=Bp, K=K),
        grid=(B // Bp,),
        in_specs=[
            pl.BlockSpec((Bp, T), lambda i: (i, 0)),
            pl.BlockSpec((Bp, K, D), lambda i: (i, 0, 0)),
            pl.BlockSpec((Bp * K, D), lambda i: (i, 0)),
            pl.BlockSpec((Bp * K, 8), lambda i: (i, 0)),
            pl.BlockSpec((Bp * K,), lambda i: (i,)),
            pl.BlockSpec((T, D), lambda i: (0, 0)),
            pl.BlockSpec((1, D), lambda i: (0, 0)),
            pl.BlockSpec((D, D), lambda i: (0, 0)),
            pl.BlockSpec((1, D), lambda i: (0, 0)),
            pl.BlockSpec((D, D), lambda i: (0, 0)),
            pl.BlockSpec((D, D), lambda i: (0, 0)),
            pl.BlockSpec((1, D), lambda i: (0, 0)),
        ],
        out_specs=[
            pl.BlockSpec((Bp, D), lambda i: (i, 0)),
            pl.BlockSpec((Bp * K,), lambda i: (i,)),
        ],
        out_shape=[jax.ShapeDtypeStruct((B, D), jnp.float32),
                   jax.ShapeDtypeStruct((B * K,), jnp.float32)],
    )(title_emb, e13, nb_rows, nnum_g.reshape(B * K, 8),
      jnp.bitwise_and(eflat, 7), W1, b1.reshape(1, D), W2,
      b2.reshape(1, D), wan, wae, ba1.reshape(1, D))

    w2c = jnp.concatenate([Wa2, Wc2], axis=1)
    b2c = jnp.concatenate([ba2, bc2]).reshape(1, 2 * D)
    zD1 = jnp.zeros((D, 1), jnp.float32)
    w3c = jnp.concatenate(
        [jnp.concatenate([Wa3, zD1], axis=1),
         jnp.concatenate([zD1, Wc3], axis=1)], axis=0)
    b3c = jnp.concatenate([ba3, bc3]).reshape(1, 2)

    Bb = 32
    act3 = act_rows.reshape(B, KK, D)
    pq = pl.pallas_call(
        functools.partial(_big_body, Bb=Bb, KK=KK, D=D),
        grid=(B // Bb,),
        in_specs=[
            pl.BlockSpec((Bb, KK, D), lambda i: (i, 0, 0)),
            pl.BlockSpec((Bb, D), lambda i: (i, 0)),
            pl.BlockSpec((D, D), lambda i: (0, 0)),
            pl.BlockSpec((D, 2 * D), lambda i: (0, 0)),
            pl.BlockSpec((1, 2 * D), lambda i: (0, 0)),
            pl.BlockSpec((2 * D, 2), lambda i: (0, 0)),
            pl.BlockSpec((1, 2), lambda i: (0, 0)),
        ],
        out_specs=pl.BlockSpec((Bb * KK, 2), lambda i: (i, 0)),
        out_shape=jax.ShapeDtypeStruct((B * KK, 2), jnp.float32),
    )(act3, s_state, wa1a, w2c, b2c, w3c, b3c)

    return (pq[:, 0:1].reshape(B, KK, 1), pq[:, 1:2].reshape(B, KK, 1),
            sim_flat.reshape(B, K))


# speedup vs baseline: 1.2627x; 1.0773x over previous
import functools

import jax
import jax.numpy as jnp
from jax import lax
from jax.experimental import pallas as pl
from jax.experimental.pallas import tpu as pltpu
from jax.experimental.pallas import tpu_sc as plsc

NW = 32
NC = 2


def _elu(x):
    return jnp.where(x > 0, x, jnp.exp(x) - 1.0)


def _sc_gather(ent, nbt, num, idx_act, idx_sml):
    NE, SD = ent.shape
    NACT = idx_act.shape[0]
    with_small = idx_sml is not None
    NSML = idx_sml.shape[0] if with_small else 0
    per_a = NACT // NW
    per_s = NSML // NW
    CA = 320
    CS = 64

    mesh = plsc.VectorSubcoreMesh(core_axis_name="c", subcore_axis_name="s")

    out_type = [jax.ShapeDtypeStruct((NACT, SD), jnp.float32)]
    scratch = [
        pltpu.VMEM((2, CA, SD), jnp.float32),
        pltpu.VMEM((per_a,), jnp.int32),
        pltpu.SemaphoreType.DMA,
        pltpu.SemaphoreType.DMA,
        pltpu.SemaphoreType.DMA,
    ]
    if with_small:
        out_type += [
            jax.ShapeDtypeStruct((NSML, SD), jnp.float32),
            jax.ShapeDtypeStruct((NSML, SD), jnp.float32),
            jax.ShapeDtypeStruct((NSML * 8,), jnp.float32),
        ]
        scratch += [
            pltpu.VMEM((2, CS, SD), jnp.float32),
            pltpu.VMEM((2, CS * 8), jnp.float32),
            pltpu.VMEM((per_s,), jnp.int32),
        ]

    @functools.partial(
        pl.kernel,
        out_type=out_type,
        mesh=mesh,
        compiler_params=pltpu.CompilerParams(needs_layout_passes=False),
        scratch_types=scratch,
    )
    def k(*args):
        if with_small:
            (ent_hbm, nbt_hbm, num_hbm, idxa_hbm, idxs_hbm,
             out_a, out_e, out_n, out_q,
             astage, aidx, semg0, semg1, sems, sstage, nstage, sidx) = args
        else:
            (ent_hbm, idxa_hbm, out_a,
             astage, aidx, semg0, semg1, sems) = args
        wid = lax.axis_index("s") * NC + lax.axis_index("c")
        abase = wid * per_a
        pltpu.sync_copy(idxa_hbm.at[pl.ds(abase, per_a)], aidx)
        if with_small:
            sbase = wid * per_s
            pltpu.sync_copy(idxs_hbm.at[pl.ds(sbase, per_s)], sidx)
        iota16 = lax.iota(jnp.int32, 16)
        gsem = (semg0, semg1)

        def phase(src, idxref, outref, hbase, nch, cp, stg, seg):
            rw = 8 if seg else src.shape[1] if len(src.shape) > 1 else 1

            def fire(g, b):
                def fire16(j, carry):
                    j16 = pl.multiple_of(g * cp + j * 16, 16)
                    vec = idxref[pl.ds(j16, 16)]
                    for l in range(16):
                        r = jnp.sum(jnp.where(iota16 == l, vec, 0))
                        if seg:
                            r8 = pl.multiple_of(jnp.bitwise_and(r, -8), 8)
                            pltpu.async_copy(
                                src.at[pl.ds(r8, 8)],
                                stg.at[b, pl.ds((j * 16 + l) * 8, 8)],
                                gsem[b])
                        else:
                            pltpu.async_copy(
                                src.at[pl.ds(r, 1)],
                                stg.at[b, pl.ds(j * 16 + l, 1)], gsem[b])
                    return carry

                lax.fori_loop(0, cp // 16, fire16, 0)

            def drain(b):
                pltpu.make_async_copy(
                    src.at[pl.ds(0, cp)] if not seg
                    else src.at[pl.ds(0, cp * 8)], stg.at[b], gsem[b]).wait()

            def store(g, b):
                if seg:
                    dst = outref.at[pl.ds((hbase + g * cp) * 8, cp * 8)]
                else:
                    dst = outref.at[pl.ds(hbase + g * cp, cp)]
                pltpu.async_copy(stg.at[b], dst, sems)
                return cp * rw * 4

            def drain_store():
                pltpu.make_async_copy(
                    src.at[pl.ds(0, cp)] if not seg
                    else src.at[pl.ds(0, cp * 8)], stg.at[0], sems).wait()

            def pair(h, carry):
                g0 = h * 2
                fire(g0, 0)
                fire(g0 + 1, 1)
                drain(0)
                store(g0, 0)
                drain(1)
                store(g0 + 1, 1)
                drain_store()
                drain_store()
                return carry

            lax.fori_loop(0, nch // 2, pair, 0)

        phase(ent_hbm, aidx, out_a, abase, per_a // CA, CA, astage, 0)
        if with_small:
            phase(ent_hbm, sidx, out_e, sbase, per_s // CS, CS, sstage, 0)
            phase(nbt_hbm, sidx, out_n, sbase, per_s // CS, CS, sstage, 0)
            phase(num_hbm, sidx, out_q, sbase, per_s // CS, CS, nstage, 1)

    if with_small:
        return k(ent, nbt, num, idx_act, idx_sml)
    return k(ent, idx_act)


def _prep_body(title_ref, e13_ref, nb2_ref, nq_ref, col_ref, W1_ref, b1_ref,
               W2_ref, b2_ref, wan_ref, wae_ref, ba1_ref, s_out, sim_out,
               *, Bp, K):
    title = title_ref[...]
    h = _elu(jnp.dot(title, W1_ref[...], preferred_element_type=jnp.float32)
             + b1_ref[...])
    news = jnp.tanh(jnp.dot(h, W2_ref[...], preferred_element_type=jnp.float32)
                    + b2_ref[...])
    me = jnp.mean(e13_ref[...], axis=1)
    s_out[...] = (jnp.dot(news, wan_ref[...], preferred_element_type=jnp.float32)
                  + jnp.dot(me, wae_ref[...], preferred_element_type=jnp.float32)
                  + ba1_ref[...])
    D = news.shape[1]
    news_exp = jnp.broadcast_to(news[:, None, :], (Bp, K, D)).reshape(Bp * K, D)
    nb = nb2_ref[...]
    diff = nb - news_exp
    dots = jnp.sum(diff * news_exp, axis=-1)
    v2 = jnp.sum(diff * diff, axis=-1)
    n2 = jnp.sum(news_exp * news_exp, axis=-1)
    cols = col_ref[...]
    onehot = (lax.broadcasted_iota(jnp.int32, (Bp * K, 8), 1)
              == cols[:, None]).astype(jnp.float32)
    nnum = jnp.sum(nq_ref[...] * onehot, axis=-1)
    na = jnp.sqrt(n2)
    nbn = jnp.sqrt(v2) / nnum
    sim_out[...] = (dots / nnum) / jnp.maximum(na * nbn, 1e-8)


def _big_body(act_ref, s_ref, wa1a_ref, w2c_ref, b2c_ref, w3c_ref, b3c_ref,
              pq_ref, *, Bb, KK, D):
    a = act_ref[...].reshape(Bb * KK, D)
    z = jnp.dot(a, wa1a_ref[...], preferred_element_type=jnp.float32)
    z = z.reshape(Bb, KK, D) + s_ref[...][:, None, :]
    ax = _elu(z).reshape(Bb * KK, D)
    uv = _elu(jnp.dot(ax, w2c_ref[...], preferred_element_type=jnp.float32)
              + b2c_ref[...])
    pq_ref[...] = jax.nn.sigmoid(
        jnp.dot(uv, w3c_ref[...], preferred_element_type=jnp.float32)
        + b3c_ref[...])


def kernel(title_emb, entity_ids, neighbor_ids, entity_table, neibor_table,
           neibor_num, W1, b1, W2, b2, Wa1, ba1, Wa2, ba2, Wa3, ba3, Wc2, bc2,
           Wc3, bc3):
    B, K = entity_ids.shape
    KK = K * K
    D = entity_table.shape[1]

    eflat = entity_ids.reshape(-1).astype(jnp.int32)
    nflat = neighbor_ids.reshape(-1).astype(jnp.int32)

    H = B // 2
    act_rows1, e1_rows, nb_rows, nnum_g = _sc_gather(
        entity_table, neibor_table, neibor_num, nflat[:H * KK], eflat)
    (act_rows2,) = _sc_gather(
        entity_table, None, None, nflat[H * KK:], None)

    wan, wae, wa1a = Wa1[:D], Wa1[D:2 * D], Wa1[2 * D:]

    Bp = 256
    T = title_emb.shape[1]
    e13 = e1_rows.reshape(B, K, D)
    s_state, sim_flat = pl.pallas_call(
        functools.partial(_prep_body, Bp=Bp, K=K),
        grid=(B // Bp,),
        in_specs=[
            pl.BlockSpec((Bp, T), lambda i: (i, 0)),
            pl.BlockSpec((Bp, K, D), lambda i: (i, 0, 0)),
            pl.BlockSpec((Bp * K, D), lambda i: (i, 0)),
            pl.BlockSpec((Bp * K, 8), lambda i: (i, 0)),
            pl.BlockSpec((Bp * K,), lambda i: (i,)),
            pl.BlockSpec((T, D), lambda i: (0, 0)),
            pl.BlockSpec((1, D), lambda i: (0, 0)),
            pl.BlockSpec((D, D), lambda i: (0, 0)),
            pl.BlockSpec((1, D), lambda i: (0, 0)),
            pl.BlockSpec((D, D), lambda i: (0, 0)),
            pl.BlockSpec((D, D), lambda i: (0, 0)),
            pl.BlockSpec((1, D), lambda i: (0, 0)),
        ],
        out_specs=[
            pl.BlockSpec((Bp, D), lambda i: (i, 0)),
            pl.BlockSpec((Bp * K,), lambda i: (i,)),
        ],
        out_shape=[jax.ShapeDtypeStruct((B, D), jnp.float32),
                   jax.ShapeDtypeStruct((B * K,), jnp.float32)],
    )(title_emb, e13, nb_rows, nnum_g.reshape(B * K, 8),
      jnp.bitwise_and(eflat, 7), W1, b1.reshape(1, D), W2,
      b2.reshape(1, D), wan, wae, ba1.reshape(1, D))

    w2c = jnp.concatenate([Wa2, Wc2], axis=1)
    b2c = jnp.concatenate([ba2, bc2]).reshape(1, 2 * D)
    zD1 = jnp.zeros((D, 1), jnp.float32)
    w3c = jnp.concatenate(
        [jnp.concatenate([Wa3, zD1], axis=1),
         jnp.concatenate([zD1, Wc3], axis=1)], axis=0)
    b3c = jnp.concatenate([ba3, bc3]).reshape(1, 2)

    Bb = 32

    def big(act_part, s_part):
        Bh = s_part.shape[0]
        act3 = act_part.reshape(Bh, KK, D)
        return pl.pallas_call(
            functools.partial(_big_body, Bb=Bb, KK=KK, D=D),
            grid=(Bh // Bb,),
            in_specs=[
                pl.BlockSpec((Bb, KK, D), lambda i: (i, 0, 0)),
                pl.BlockSpec((Bb, D), lambda i: (i, 0)),
                pl.BlockSpec((D, D), lambda i: (0, 0)),
                pl.BlockSpec((D, 2 * D), lambda i: (0, 0)),
                pl.BlockSpec((1, 2 * D), lambda i: (0, 0)),
                pl.BlockSpec((2 * D, 2), lambda i: (0, 0)),
                pl.BlockSpec((1, 2), lambda i: (0, 0)),
            ],
            out_specs=pl.BlockSpec((Bb * KK, 2), lambda i: (i, 0)),
            out_shape=jax.ShapeDtypeStruct((Bh * KK, 2), jnp.float32),
        )(act3, s_part, wa1a, w2c, b2c, w3c, b3c)

    pq1 = big(act_rows1, s_state[:H])
    pq2 = big(act_rows2, s_state[H:])
    pq = jnp.concatenate([pq1, pq2], axis=0)

    return (pq[:, 0:1].reshape(B, KK, 1), pq[:, 1:2].reshape(B, KK, 1),
            sim_flat.reshape(B, K))
